# Initial kernel scaffold; baseline (speedup 1.0000x reference)
import jax
import jax.numpy as jnp
from jax.experimental import pallas as pl


def kernel(x, weights, or_padding_mask):
    raise NotImplementedError("write your pallas kernel here")


# speedup vs baseline: 15.8502x; 15.8502x over previous
import functools

import jax
import jax.numpy as jnp
from jax import lax
from jax.experimental import pallas as pl

B, F = 2048, 512
OUT, OR_T, AND_T = 1024, 8, 4
C = OUT * OR_T


def _prep_kernel(wk_ref, mask_ref, d_ref, e_ref):
    cc = d_ref.shape[1]
    iota = lax.broadcasted_iota(jnp.int32, (F, cc), 0)
    pos = iota + 1
    neg = iota + (F + 1)
    d = jnp.zeros((F, cc), jnp.float32)
    e = jnp.zeros((1, cc), jnp.float32)
    for k in range(AND_T):
        wk = wk_ref[k : k + 1, :]
        d = d + (wk == pos).astype(jnp.float32) - (wk == neg).astype(jnp.float32)
        e = e + ((wk == 0) | (wk > F)).astype(jnp.float32)
    d_ref[...] = d.astype(jnp.bfloat16)
    e_ref[...] = jnp.where(mask_ref[...] != 0, e, -1000.0)


def _main_kernel(x_ref, d_ref, e_ref, o_ref):
    xb = (x_ref[...] != 0.0).astype(jnp.bfloat16)
    s = jnp.dot(xb, d_ref[...], preferred_element_type=jnp.float32)
    s = s + e_ref[...]
    m = s[:, 0:OUT]
    for t in range(1, OR_T):
        m = jnp.maximum(m, s[:, t * OUT : (t + 1) * OUT])
    o_ref[...] = (m >= 3.5).astype(jnp.int8)


@jax.jit
def kernel(x, weights, or_padding_mask):
    wk = weights.transpose(2, 1, 0).reshape(AND_T, C)
    mask = or_padding_mask.transpose(1, 0).reshape(1, C).astype(jnp.int32)

    cc = 2048
    d, e = pl.pallas_call(
        _prep_kernel,
        grid=(C // cc,),
        in_specs=[
            pl.BlockSpec((AND_T, cc), lambda j: (0, j)),
            pl.BlockSpec((1, cc), lambda j: (0, j)),
        ],
        out_specs=[
            pl.BlockSpec((F, cc), lambda j: (0, j)),
            pl.BlockSpec((1, cc), lambda j: (0, j)),
        ],
        out_shape=[
            jax.ShapeDtypeStruct((F, C), jnp.bfloat16),
            jax.ShapeDtypeStruct((1, C), jnp.float32),
        ],
    )(wk, mask)

    bb = 256
    out_i8 = pl.pallas_call(
        _main_kernel,
        grid=(B // bb,),
        in_specs=[
            pl.BlockSpec((bb, F), lambda i: (i, 0)),
            pl.BlockSpec((F, C), lambda i: (0, 0)),
            pl.BlockSpec((1, C), lambda i: (0, 0)),
        ],
        out_specs=pl.BlockSpec((bb, OUT), lambda i: (i, 0)),
        out_shape=jax.ShapeDtypeStruct((B, OUT), jnp.int8),
    )(x, d, e)

    return out_i8.astype(jnp.bool_)


# speedup vs baseline: 15.9589x; 1.0069x over previous
import functools

import jax
import jax.numpy as jnp
from jax import lax
from jax.experimental import pallas as pl

B, F = 2048, 512
OUT, OR_T, AND_T = 1024, 8, 4
C = OUT * OR_T


def _prep_kernel(wk_ref, mask_ref, d_ref, e_ref):
    cc = d_ref.shape[1]
    iota = lax.broadcasted_iota(jnp.int32, (F, cc), 0)
    pos = iota + 1
    neg = iota + (F + 1)
    d = jnp.zeros((F, cc), jnp.float32)
    e = jnp.zeros((1, cc), jnp.float32)
    for k in range(AND_T):
        wk = wk_ref[k : k + 1, :]
        d = d + (wk == pos).astype(jnp.float32) - (wk == neg).astype(jnp.float32)
        e = e + ((wk == 0) | (wk > F)).astype(jnp.float32)
    d_ref[...] = d.astype(jnp.bfloat16)
    e_ref[...] = jnp.where(mask_ref[...] != 0, e, -1000.0).astype(jnp.bfloat16)


def _main_kernel(x_ref, d_ref, e_ref, o_ref):
    xb = (x_ref[...] != 0.0).astype(jnp.bfloat16)
    s = jnp.dot(xb, d_ref[...], preferred_element_type=jnp.float32)
    s = s + e_ref[...]
    m = s[:, 0:OUT]
    for t in range(1, OR_T):
        m = jnp.maximum(m, s[:, t * OUT : (t + 1) * OUT])
    o_ref[...] = (m >= 3.5).astype(jnp.int8)


@jax.jit
def kernel(x, weights, or_padding_mask):
    wk = weights.transpose(2, 1, 0).reshape(AND_T, C)
    mask = or_padding_mask.transpose(1, 0).reshape(1, C).astype(jnp.int32)

    cc = 2048
    d, e = pl.pallas_call(
        _prep_kernel,
        grid=(C // cc,),
        in_specs=[
            pl.BlockSpec((AND_T, cc), lambda j: (0, j)),
            pl.BlockSpec((1, cc), lambda j: (0, j)),
        ],
        out_specs=[
            pl.BlockSpec((F, cc), lambda j: (0, j)),
            pl.BlockSpec((1, cc), lambda j: (0, j)),
        ],
        out_shape=[
            jax.ShapeDtypeStruct((F, C), jnp.bfloat16),
            jax.ShapeDtypeStruct((1, C), jnp.bfloat16),
        ],
    )(wk, mask)

    bb = 512
    out_i8 = pl.pallas_call(
        _main_kernel,
        grid=(B // bb,),
        in_specs=[
            pl.BlockSpec((bb, F), lambda i: (i, 0)),
            pl.BlockSpec((F, C), lambda i: (0, 0)),
            pl.BlockSpec((1, C), lambda i: (0, 0)),
        ],
        out_specs=pl.BlockSpec((bb, OUT), lambda i: (i, 0)),
        out_shape=jax.ShapeDtypeStruct((B, OUT), jnp.int8),
    )(x, d, e)

    return out_i8.astype(jnp.bool_)


# speedup vs baseline: 20.0788x; 1.2582x over previous
import jax
import jax.numpy as jnp
from jax import lax
from jax.experimental import pallas as pl
from jax.experimental.pallas import tpu as pltpu

B, F = 2048, 512
OUT, OR_T, AND_T = 1024, 8, 4
C = OUT * OR_T


def _fused_kernel(wk_ref, mask_ref, x_ref, o_ref, d_s, e_s):
    @pl.when(pl.program_id(0) == 0)
    def _prep():
        iota = lax.broadcasted_iota(jnp.int32, (F, C), 0)
        d = jnp.zeros((F, C), jnp.bfloat16)
        e = jnp.zeros((1, C), jnp.float32)
        for k in range(AND_T):
            wk = wk_ref[k : k + 1, :]
            q = (wk - 1) & (F - 1)
            sgn_i = (wk >= 1).astype(jnp.int32) * (1 - 2 * (wk > F).astype(jnp.int32))
            d = d + (q == iota).astype(jnp.bfloat16) * sgn_i.astype(jnp.bfloat16)
            e = e + (wk == 0).astype(jnp.float32) + (wk > F).astype(jnp.float32)
        d_s[...] = d
        e_s[...] = jnp.where(mask_ref[...] != 0, e, -1000.0).astype(jnp.bfloat16)

    xb = (x_ref[...] != 0.0).astype(jnp.bfloat16)
    s = jnp.dot(xb, d_s[...], preferred_element_type=jnp.float32)
    s = s + e_s[...].astype(jnp.float32)
    m = s[:, 0:OUT]
    for t in range(1, OR_T):
        m = jnp.maximum(m, s[:, t * OUT : (t + 1) * OUT])
    o_ref[...] = (m >= 3.5).astype(jnp.int8)


@jax.jit
def kernel(x, weights, or_padding_mask):
    wk = weights.transpose(2, 1, 0).reshape(AND_T, C)
    mask = or_padding_mask.transpose(1, 0).reshape(1, C).astype(jnp.int32)

    bb = 512
    out_i8 = pl.pallas_call(
        _fused_kernel,
        grid=(B // bb,),
        in_specs=[
            pl.BlockSpec((AND_T, C), lambda i: (0, 0)),
            pl.BlockSpec((1, C), lambda i: (0, 0)),
            pl.BlockSpec((bb, F), lambda i: (i, 0)),
        ],
        out_specs=pl.BlockSpec((bb, OUT), lambda i: (i, 0)),
        out_shape=jax.ShapeDtypeStruct((B, OUT), jnp.int8),
        scratch_shapes=[
            pltpu.VMEM((F, C), jnp.bfloat16),
            pltpu.VMEM((1, C), jnp.bfloat16),
        ],
    )(wk, mask, x)

    return out_i8.astype(jnp.bool_)
